# baseline (speedup 1.0000x reference)
import functools

import jax
import jax.numpy as jnp
from jax.experimental import pallas as pl
from jax.experimental.pallas import tpu as pltpu


def _se_kernel(x_ref, w1_ref, w2t_ref, b2_ref, o_ref, *, inv_hw):
    x = x_ref[...]
    pooled = jnp.sum(x, axis=(1, 2)) * inv_hw
    h = jnp.maximum(
        jax.lax.dot_general(pooled, w1_ref[...],
                            (((1,), (1,)), ((), ())),
                            preferred_element_type=jnp.float32), 0.0)
    z = jnp.dot(h, w2t_ref[...],
                preferred_element_type=jnp.float32)
    gate = jax.nn.sigmoid(z + b2_ref[...])
    o_ref[...] = x * gate[:, None, None, :]


def kernel(x, w1, w2, b2):
    N, C, H, W = x.shape
    Cr = w1.shape[0]

    x_nhwc = x.transpose(0, 2, 3, 1)
    w2t = w2.T
    b2_row = b2.reshape(1, C)

    nb = 4
    while N % nb:
        nb //= 2
    steps = N // nb
    blk = (nb, H, W, C)

    body = functools.partial(_se_kernel, inv_hw=1.0 / (H * W))
    out_nhwc = pl.pallas_call(
        body,
        out_shape=jax.ShapeDtypeStruct((N, H, W, C), x.dtype),
        grid=(steps,),
        in_specs=[
            pl.BlockSpec(blk, lambda n: (n, 0, 0, 0)),
            pl.BlockSpec(w1.shape, lambda n: (0, 0)),
            pl.BlockSpec(w2t.shape, lambda n: (0, 0)),
            pl.BlockSpec(b2_row.shape, lambda n: (0, 0)),
        ],
        out_specs=pl.BlockSpec(blk, lambda n: (n, 0, 0, 0)),
        compiler_params=pltpu.CompilerParams(
            dimension_semantics=("parallel",),
            vmem_limit_bytes=64 << 20,
        ),
        cost_estimate=pl.CostEstimate(
            flops=int(N * (2 * C * H * W + 4 * C * Cr + 3 * C)),
            transcendentals=int(N * C),
            bytes_accessed=int(2 * N * C * H * W * 4),
        ),
    )(x_nhwc, w1, w2t, b2_row)
    return out_nhwc.transpose(0, 3, 1, 2)


# speedup vs baseline: 1.2371x; 1.2371x over previous
import functools

import jax
import jax.numpy as jnp
from jax.experimental import pallas as pl
from jax.experimental.pallas import tpu as pltpu


def _se_kernel(x_ref, w1_ref, w2t_ref, b2_ref, o_ref, *, inv_hw):
    x = x_ref[...]
    pooled = jnp.sum(x, axis=(1, 2)) * inv_hw
    h = jnp.maximum(
        jax.lax.dot_general(pooled, w1_ref[...],
                            (((1,), (1,)), ((), ())),
                            preferred_element_type=jnp.float32), 0.0)
    z = jnp.dot(h, w2t_ref[...],
                preferred_element_type=jnp.float32)
    gate = jax.nn.sigmoid(z + b2_ref[...])
    o_ref[...] = x * gate[:, None, None, :]


def kernel(x, w1, w2, b2):
    N, C, H, W = x.shape
    Cr = w1.shape[0]

    x_nhwc = x.transpose(0, 2, 3, 1)
    w2t = w2.T
    b2_row = b2.reshape(1, C)

    nb = 16
    while N % nb:
        nb //= 2
    steps = N // nb
    blk = (nb, H, W, C)

    body = functools.partial(_se_kernel, inv_hw=1.0 / (H * W))
    out_nhwc = pl.pallas_call(
        body,
        out_shape=jax.ShapeDtypeStruct((N, H, W, C), x.dtype),
        grid=(steps,),
        in_specs=[
            pl.BlockSpec(blk, lambda n: (n, 0, 0, 0)),
            pl.BlockSpec(w1.shape, lambda n: (0, 0)),
            pl.BlockSpec(w2t.shape, lambda n: (0, 0)),
            pl.BlockSpec(b2_row.shape, lambda n: (0, 0)),
        ],
        out_specs=pl.BlockSpec(blk, lambda n: (n, 0, 0, 0)),
        compiler_params=pltpu.CompilerParams(
            dimension_semantics=("parallel",),
            vmem_limit_bytes=64 << 20,
        ),
        cost_estimate=pl.CostEstimate(
            flops=int(N * (2 * C * H * W + 4 * C * Cr + 3 * C)),
            transcendentals=int(N * C),
            bytes_accessed=int(2 * N * C * H * W * 4),
        ),
    )(x_nhwc, w1, w2t, b2_row)
    return out_nhwc.transpose(0, 3, 1, 2)
